# baseline (speedup 1.0000x reference)
import functools

import jax
import jax.numpy as jnp
from jax import lax
from jax.experimental import pallas as pl
from jax.experimental.pallas import tpu as pltpu
from jax.experimental.pallas import tpu_sc as plsc

_DEBUG_XLA_DECODER = False
_MSG_STAGE = 3
_DEBUG_XLA_MSG = False

N = 10000
H = 256
HH = 128
E = 160000
EL = 160000
CH = 128
NTILES = 16
NWORK = 32

EP = 163840
MSG_CHUNKS_PER_TILE = 80
ELP = 163840
DEC_CHUNKS_PER_WORKER = 40
ACC_ROWS = N + 8

WB_ROWS = 624
WB_REM = N - NTILES * WB_ROWS


def _mesh():
    return plsc.VectorSubcoreMesh(core_axis_name="c", subcore_axis_name="s")


_SC_TEST = 0


@functools.partial(
    pl.kernel,
    out_type=jax.ShapeDtypeStruct((N, HH), jnp.float32),
    mesh=_mesh(),
    scratch_types=[pltpu.VMEM((312, HH), jnp.float32)])
def _sc_ident(x_hbm, out_hbm, buf):
    c = lax.axis_index("c")
    s = lax.axis_index("s")
    w = s * 2 + c
    b = w * 312
    pltpu.sync_copy(x_hbm.at[pl.ds(b, 312)], buf)
    pltpu.sync_copy(buf, out_hbm.at[pl.ds(b, 312)])

    @pl.when(w == 0)
    def _():
        pltpu.sync_copy(x_hbm.at[pl.ds(9984, 16)], buf.at[pl.ds(0, 16)])
        pltpu.sync_copy(buf.at[pl.ds(0, 16)], out_hbm.at[pl.ds(9984, 16)])


@functools.partial(
    pl.kernel,
    out_type=jax.ShapeDtypeStruct((N, HH), jnp.float32),
    mesh=_mesh(),
    scratch_types=[pltpu.VMEM_SHARED((ACC_ROWS, HH), jnp.float32)])
def _sc_ident2(x_hbm, out_hbm, acc_sh):
    c = lax.axis_index("c")
    s = lax.axis_index("s")
    ib = s * WB_ROWS
    rem0 = NTILES * WB_ROWS
    pltpu.sync_copy(x_hbm.at[pl.ds(ib, WB_ROWS)],
                    acc_sh.at[pl.ds(ib, WB_ROWS)])

    @pl.when(s == 0)
    def _():
        pltpu.sync_copy(x_hbm.at[pl.ds(rem0, WB_REM)],
                        acc_sh.at[pl.ds(rem0, WB_REM)])

    plsc.subcore_barrier()
    if _SC_TEST == 4:
        plsc.subcore_barrier()

    @pl.when(c == 0)
    def _():
        pltpu.sync_copy(acc_sh.at[pl.ds(ib, WB_ROWS)],
                        out_hbm.at[pl.ds(ib, WB_ROWS)])

        @pl.when(s == 0)
        def _():
            pltpu.sync_copy(acc_sh.at[pl.ds(rem0, WB_REM)],
                            out_hbm.at[pl.ds(rem0, WB_REM)])


def _make_msg(W):
    out_type = (jax.ShapeDtypeStruct((N, W), jnp.float32),
                jax.ShapeDtypeStruct((N, W), jnp.float32))
    scratch = [
        pltpu.VMEM((CH,), jnp.int32),
        pltpu.VMEM((CH,), jnp.int32),
        pltpu.VMEM((CH, W), jnp.float32),
        pltpu.VMEM_SHARED((ACC_ROWS, W), jnp.float32),
        pltpu.SemaphoreType.DMA,
    ]

    @functools.partial(pl.kernel, out_type=out_type, mesh=_mesh(),
                       scratch_types=scratch)
    def msg(x_lo, x_hi, src_hbm, dst_hbm, z_acc, out_lo, out_hi,
            src_c, dst_c, rows_v, acc_sh, sem):
        c = lax.axis_index("c")
        s = lax.axis_index("s")

        ib = s * WB_ROWS
        rem0 = NTILES * WB_ROWS
        nrem = ACC_ROWS - rem0
        pltpu.sync_copy(z_acc.at[pl.ds(ib, WB_ROWS)],
                        acc_sh.at[pl.ds(ib, WB_ROWS)])

        @pl.when(s == 0)
        def _():
            pltpu.sync_copy(z_acc.at[pl.ds(rem0, nrem)],
                            acc_sh.at[pl.ds(rem0, nrem)])

        base = s * MSG_CHUNKS_PER_TILE
        plsc.subcore_barrier()

        def run(x_ref):
            def body(i, carry):
                pltpu.sync_copy(src_hbm.at[base + i], src_c)
                pltpu.sync_copy(dst_hbm.at[base + i], dst_c)
                if _MSG_STAGE >= 2:
                    pltpu.async_copy(x_ref.at[src_c], rows_v, sem).wait()
                if _MSG_STAGE >= 3:
                    pltpu.sync_copy(rows_v, acc_sh.at[dst_c], add=True)
                return carry
            if _MSG_STAGE >= 1:
                lax.fori_loop(0, MSG_CHUNKS_PER_TILE, body, 0)

        @pl.when(c == 0)
        def _():
            run(x_lo)

        @pl.when(c == 1)
        def _():
            run(x_hi)

        plsc.subcore_barrier()

        @pl.when(c == 0)
        def _():
            pltpu.sync_copy(acc_sh.at[pl.ds(ib, WB_ROWS)],
                            out_lo.at[pl.ds(ib, WB_ROWS)])

            @pl.when(s == 0)
            def _():
                pltpu.sync_copy(acc_sh.at[pl.ds(rem0, WB_REM)],
                                out_lo.at[pl.ds(rem0, WB_REM)])

        @pl.when(c == 1)
        def _():
            pltpu.sync_copy(acc_sh.at[pl.ds(ib, WB_ROWS)],
                            out_hi.at[pl.ds(ib, WB_ROWS)])

            @pl.when(s == 0)
            def _():
                pltpu.sync_copy(acc_sh.at[pl.ds(rem0, WB_REM)],
                                out_hi.at[pl.ds(rem0, WB_REM)])

    return msg


_msg_half = _make_msg(HH)


@functools.partial(
    pl.kernel,
    out_type=jax.ShapeDtypeStruct((N, HH), jnp.float32),
    mesh=_mesh(),
    scratch_types=[
        pltpu.VMEM((CH,), jnp.int32),
        pltpu.VMEM((CH, HH), jnp.float32),
        pltpu.VMEM_SHARED((ACC_ROWS, HH), jnp.float32),
    ])
def _counts(ones_tbl, dst_hbm, z_acc, out, dst_c, rows_v, acc_sh):
    c = lax.axis_index("c")
    s = lax.axis_index("s")
    ib = s * WB_ROWS
    rem0 = NTILES * WB_ROWS
    nrem = ACC_ROWS - rem0
    base = s * MSG_CHUNKS_PER_TILE

    @pl.when(c == 0)
    def _():
        pltpu.sync_copy(z_acc.at[pl.ds(ib, WB_ROWS)],
                        acc_sh.at[pl.ds(ib, WB_ROWS)])

        @pl.when(s == 0)
        def _():
            pltpu.sync_copy(z_acc.at[pl.ds(rem0, nrem)],
                            acc_sh.at[pl.ds(rem0, nrem)])
        pltpu.sync_copy(ones_tbl, rows_v)

    plsc.subcore_barrier()

    @pl.when(c == 0)
    def _():
        def body(i, carry):
            pltpu.sync_copy(dst_hbm.at[base + i], dst_c)
            pltpu.sync_copy(rows_v, acc_sh.at[dst_c], add=True)
            return carry
        lax.fori_loop(0, MSG_CHUNKS_PER_TILE, body, 0)

    plsc.subcore_barrier()

    @pl.when(c == 0)
    def _():
        pltpu.sync_copy(acc_sh.at[pl.ds(ib, WB_ROWS)],
                        out.at[pl.ds(ib, WB_ROWS)])

        @pl.when(s == 0)
        def _():
            pltpu.sync_copy(acc_sh.at[pl.ds(rem0, WB_REM)],
                            out.at[pl.ds(rem0, WB_REM)])


@functools.partial(
    pl.kernel,
    out_type=jax.ShapeDtypeStruct((ELP * 16,), jnp.float32),
    mesh=_mesh(),
    scratch_types=[
        pltpu.VMEM((CH,), jnp.int32),
        pltpu.VMEM((CH,), jnp.int32),
        pltpu.VMEM((CH, H), jnp.float32),
        pltpu.VMEM((CH, H), jnp.float32),
        pltpu.VMEM((CH * 16,), jnp.float32),
        pltpu.VMEM((H,), jnp.float32),
        pltpu.SemaphoreType.DMA,
        pltpu.SemaphoreType.DMA,
    ])
def _decoder(pu_hbm, pb_hbm, row_hbm, col_hbm, w2_hbm, out_hbm,
             row_c, col_c, pur_v, pbr_v, part_v, w2_v, sem1, sem2):
    c = lax.axis_index("c")
    s = lax.axis_index("s")
    w = s * 2 + c

    pltpu.sync_copy(w2_hbm, w2_v)
    base = w * DEC_CHUNKS_PER_WORKER

    w2c = [w2_v[pl.ds(16 * j, 16)] for j in range(16)]

    def chunk(i, carry):
        pltpu.sync_copy(row_hbm.at[base + i], row_c)
        pltpu.sync_copy(col_hbm.at[base + i], col_c)
        cp1 = pltpu.async_copy(pu_hbm.at[row_c], pur_v, sem1)
        cp2 = pltpu.async_copy(pb_hbm.at[col_c], pbr_v, sem2)
        cp1.wait()
        cp2.wait()

        def edge(e, carry2):
            acc = jnp.zeros((16,), jnp.float32)
            for j in range(16):
                a = pur_v[e, pl.ds(16 * j, 16)]
                b = pbr_v[e, pl.ds(16 * j, 16)]
                acc = acc + jnp.maximum(a + b, 0.0) * w2c[j]
            part_v[pl.ds(e * 16, 16)] = acc
            return carry2

        lax.fori_loop(0, CH, edge, 0)
        pltpu.sync_copy(part_v, out_hbm.at[pl.ds((base + i) * CH * 16, CH * 16)])
        return carry

    lax.fori_loop(0, DEC_CHUNKS_PER_WORKER, chunk, 0)


_R = 1000


def _dot(a, b):
    return jnp.dot(a, b, preferred_element_type=jnp.float32)


def _proj_body(ux_ref, bx_ref, wu_ref, bu_ref, wb_ref, bb_ref,
               xul, xuh, xbl, xbh):
    xu = _dot(ux_ref[...], wu_ref[...]) + bu_ref[...]
    xb = _dot(bx_ref[...], wb_ref[...]) + bb_ref[...]
    xul[...] = xu[:, :HH]
    xuh[...] = xu[:, HH:]
    xbl[...] = xb[:, :HH]
    xbh[...] = xb[:, HH:]


def _tc_proj(ux, bx, wu, bu, wb, bb):
    return pl.pallas_call(
        _proj_body,
        grid=(N // _R,),
        in_specs=[
            pl.BlockSpec((_R, 8), lambda i: (i, 0)),
            pl.BlockSpec((_R, 384), lambda i: (i, 0)),
            pl.BlockSpec((8, H), lambda i: (0, 0)),
            pl.BlockSpec((1, H), lambda i: (0, 0)),
            pl.BlockSpec((384, H), lambda i: (0, 0)),
            pl.BlockSpec((1, H), lambda i: (0, 0)),
        ],
        out_specs=[pl.BlockSpec((_R, HH), lambda i: (i, 0))] * 4,
        out_shape=[jax.ShapeDtypeStruct((N, HH), jnp.float32)] * 4,
    )(ux, bx, wu, bu, wb, bb)


def _make_combine_body(relu, proj):
    def body(sl, sh, cnt, xl, xh, wll, wlh, wrl, wrh, b, *rest):
        inv = 1.0 / jnp.maximum(cnt[...][:, :1], 1.0)
        h = (_dot(sl[...] * inv, wll[...]) + _dot(sh[...] * inv, wlh[...])
             + _dot(xl[...], wrl[...]) + _dot(xh[...], wrh[...]) + b[...])
        if relu:
            h = jnp.maximum(h, 0.0)
        if proj:
            wp, bp, pout = rest
            pout[...] = _dot(h, wp[...]) + bp[...]
        else:
            ol, oh = rest
            ol[...] = h[:, :HH]
            oh[...] = h[:, HH:]
    return body


_combine1_body = _make_combine_body(relu=True, proj=False)
_combine2_body = _make_combine_body(relu=False, proj=True)

_w_spec = [
    pl.BlockSpec((HH, H), lambda i: (0, 0)),
    pl.BlockSpec((HH, H), lambda i: (0, 0)),
    pl.BlockSpec((HH, H), lambda i: (0, 0)),
    pl.BlockSpec((HH, H), lambda i: (0, 0)),
    pl.BlockSpec((1, H), lambda i: (0, 0)),
]
_node_half_spec = [
    pl.BlockSpec((_R, HH), lambda i: (i, 0)),
    pl.BlockSpec((_R, HH), lambda i: (i, 0)),
    pl.BlockSpec((_R, 16), lambda i: (i, 0)),
    pl.BlockSpec((_R, HH), lambda i: (i, 0)),
    pl.BlockSpec((_R, HH), lambda i: (i, 0)),
]


def _tc_combine1(sl, sh, cnt, xl, xh, wll, wlh, wrl, wrh, b):
    return pl.pallas_call(
        _combine1_body,
        grid=(N // _R,),
        in_specs=_node_half_spec + _w_spec,
        out_specs=[pl.BlockSpec((_R, HH), lambda i: (i, 0))] * 2,
        out_shape=[jax.ShapeDtypeStruct((N, HH), jnp.float32)] * 2,
    )(sl, sh, cnt, xl, xh, wll, wlh, wrl, wrh, b)


def _tc_combine2(sl, sh, cnt, xl, xh, wll, wlh, wrl, wrh, b, wp, bp):
    return pl.pallas_call(
        _combine2_body,
        grid=(N // _R,),
        in_specs=_node_half_spec + _w_spec + [
            pl.BlockSpec((H, H), lambda i: (0, 0)),
            pl.BlockSpec((1, H), lambda i: (0, 0)),
        ],
        out_specs=pl.BlockSpec((_R, H), lambda i: (i, 0)),
        out_shape=jax.ShapeDtypeStruct((N, H), jnp.float32),
    )(sl, sh, cnt, xl, xh, wll, wlh, wrl, wrh, b, wp, bp)


def _tail_body(p_ref, b2_ref, o_ref):
    o_ref[...] = jnp.sum(p_ref[...], axis=1, keepdims=True) + b2_ref[0, 0]


def _tc_tail(parts, b2):
    return pl.pallas_call(
        _tail_body,
        grid=(ELP // 4096,),
        in_specs=[pl.BlockSpec((4096, 16), lambda i: (i, 0)),
                  pl.BlockSpec((1, 1), lambda i: (0, 0),
                               memory_space=pltpu.SMEM)],
        out_specs=pl.BlockSpec((4096, 1), lambda i: (i, 0)),
        out_shape=jax.ShapeDtypeStruct((ELP, 1), jnp.float32),
    )(parts, b2)


def _pad_edges(idx, pad_len, pad_val):
    return jnp.concatenate(
        [idx, jnp.full((pad_len,), pad_val, jnp.int32)]).reshape(-1, CH)


def kernel(user_x, book_x, edge_index_ub, edge_index_bu, edge_label_index,
           user_lin_w, user_lin_b, book_lin_w, book_lin_b,
           c1_ub_wl, c1_ub_wr, c1_ub_b, c1_bu_wl, c1_bu_wr, c1_bu_b,
           c2_ub_wl, c2_ub_wr, c2_ub_b, c2_bu_wl, c2_bu_wr, c2_bu_b,
           dec_w1, dec_b1, dec_w2, dec_b2):
    f32 = jnp.float32
    ux8 = jnp.pad(user_x, ((0, 0), (0, 5)))
    wu8 = jnp.pad(user_lin_w, ((0, 5), (0, 0)))
    src_ub = _pad_edges(edge_index_ub[0], EP - E, 0)
    dst_ub = _pad_edges(edge_index_ub[1], EP - E, N)
    src_bu = _pad_edges(edge_index_bu[0], EP - E, 0)
    dst_bu = _pad_edges(edge_index_bu[1], EP - E, N)
    row_l = _pad_edges(edge_label_index[0], ELP - EL, 0)
    col_l = _pad_edges(edge_label_index[1], ELP - EL, 0)
    z_acc = jnp.zeros((ACC_ROWS, HH), f32)
    ones_tbl = jnp.zeros((CH, HH), f32).at[:, 0].set(1.0)
    w1u = dec_w1[:H]
    w1b = dec_w1[H:]
    b1r = dec_b1.reshape(1, H)
    zb = jnp.zeros((1, H), f32)

    def split(w):
        return w[:HH], w[HH:]

    xu_lo, xu_hi, xb_lo, xb_hi = _tc_proj(
        ux8, book_x, wu8, user_lin_b.reshape(1, H),
        book_lin_w, book_lin_b.reshape(1, H))
    if _SC_TEST == 1:
        xu_lo = _sc_ident(xu_lo)
    elif _SC_TEST in (2, 4):
        xu_lo = _sc_ident2(xu_lo)

    if _DEBUG_XLA_MSG:
        def _xla_msg(xl, xh, src, dst, want_counts):
            s = jax.ops.segment_sum(
                jnp.concatenate([xl, xh], axis=1)[src.reshape(-1)],
                dst.reshape(-1), num_segments=ACC_ROWS)
            outs = [s[:N, :HH], s[:N, HH:]]
            if want_counts:
                cnt = jax.ops.segment_sum(
                    jnp.ones((EP,), f32), dst.reshape(-1),
                    num_segments=ACC_ROWS)
                outs.append(jnp.broadcast_to(cnt[:N, None], (N, 16)))
            return outs
        sb_lo, sb_hi, cb = _xla_msg(xu_lo, xu_hi, src_ub, dst_ub, True)
        su_lo, su_hi, cu = _xla_msg(xb_lo, xb_hi, src_bu, dst_bu, True)
    else:
        sb_lo, sb_hi = _msg_half(xu_lo, xu_hi, src_ub, dst_ub, z_acc)
        su_lo, su_hi = _msg_half(xb_lo, xb_hi, src_bu, dst_bu, z_acc)
        cb = _counts(ones_tbl, dst_ub, z_acc)[:, :16]
        cu = _counts(ones_tbl, dst_bu, z_acc)[:, :16]
    wll, wlh = split(c1_ub_wl)
    wrl, wrh = split(c1_ub_wr)
    xb1_lo, xb1_hi = _tc_combine1(sb_lo, sb_hi, cb, xb_lo, xb_hi,
                                  wll, wlh, wrl, wrh, c1_ub_b.reshape(1, H))
    wll, wlh = split(c1_bu_wl)
    wrl, wrh = split(c1_bu_wr)
    xu1_lo, xu1_hi = _tc_combine1(su_lo, su_hi, cu, xu_lo, xu_hi,
                                  wll, wlh, wrl, wrh, c1_bu_b.reshape(1, H))

    if _DEBUG_XLA_MSG:
        sb2_lo, sb2_hi = _xla_msg(xu1_lo, xu1_hi, src_ub, dst_ub, False)
        su2_lo, su2_hi = _xla_msg(xb1_lo, xb1_hi, src_bu, dst_bu, False)
    else:
        sb2_lo, sb2_hi = _msg_half(xu1_lo, xu1_hi, src_ub, dst_ub, z_acc)
        su2_lo, su2_hi = _msg_half(xb1_lo, xb1_hi, src_bu, dst_bu, z_acc)
    wll, wlh = split(c2_ub_wl)
    wrl, wrh = split(c2_ub_wr)
    pb = _tc_combine2(sb2_lo, sb2_hi, cb, xb1_lo, xb1_hi,
                      wll, wlh, wrl, wrh, c2_ub_b.reshape(1, H), w1b, b1r)
    wll, wlh = split(c2_bu_wl)
    wrl, wrh = split(c2_bu_wr)
    pu = _tc_combine2(su2_lo, su2_hi, cu, xu1_lo, xu1_hi,
                      wll, wlh, wrl, wrh, c2_bu_b.reshape(1, H), w1u, zb)

    if _DEBUG_XLA_DECODER:
        z = jax.nn.relu(jnp.take(pu, edge_label_index[0], axis=0)
                        + jnp.take(pb, edge_label_index[1], axis=0))
        return (z @ dec_w2 + dec_b2).reshape(-1)
    parts = _decoder(pu, pb, row_l, col_l, dec_w2.reshape(H))
    out = _tc_tail(parts.reshape(ELP, 16), dec_b2.reshape(1, 1))
    return out.reshape(-1)[:EL]


# speedup vs baseline: 1.4014x; 1.4014x over previous
import functools

import jax
import jax.numpy as jnp
from jax import lax
from jax.experimental import pallas as pl
from jax.experimental.pallas import tpu as pltpu
from jax.experimental.pallas import tpu_sc as plsc

_DEBUG_XLA_DECODER = False
_MSG_STAGE = 3
_DEBUG_XLA_MSG = False

N = 10000
H = 256
HH = 128
E = 160000
EL = 160000
CH = 128
NTILES = 16
NWORK = 32

EP = 163840
MSG_CHUNKS_PER_TILE = 80
ELP = 163840
DEC_CHUNKS_PER_WORKER = 40
ACC_ROWS = N + 8

WB_ROWS = 624
WB_REM = N - NTILES * WB_ROWS


def _mesh():
    return plsc.VectorSubcoreMesh(core_axis_name="c", subcore_axis_name="s")


_SC_TEST = 0


@functools.partial(
    pl.kernel,
    out_type=jax.ShapeDtypeStruct((N, HH), jnp.float32),
    mesh=_mesh(),
    scratch_types=[pltpu.VMEM((312, HH), jnp.float32)])
def _sc_ident(x_hbm, out_hbm, buf):
    c = lax.axis_index("c")
    s = lax.axis_index("s")
    w = s * 2 + c
    b = w * 312
    pltpu.sync_copy(x_hbm.at[pl.ds(b, 312)], buf)
    pltpu.sync_copy(buf, out_hbm.at[pl.ds(b, 312)])

    @pl.when(w == 0)
    def _():
        pltpu.sync_copy(x_hbm.at[pl.ds(9984, 16)], buf.at[pl.ds(0, 16)])
        pltpu.sync_copy(buf.at[pl.ds(0, 16)], out_hbm.at[pl.ds(9984, 16)])


@functools.partial(
    pl.kernel,
    out_type=jax.ShapeDtypeStruct((N, HH), jnp.float32),
    mesh=_mesh(),
    scratch_types=[pltpu.VMEM_SHARED((ACC_ROWS, HH), jnp.float32)])
def _sc_ident2(x_hbm, out_hbm, acc_sh):
    c = lax.axis_index("c")
    s = lax.axis_index("s")
    ib = s * WB_ROWS
    rem0 = NTILES * WB_ROWS
    pltpu.sync_copy(x_hbm.at[pl.ds(ib, WB_ROWS)],
                    acc_sh.at[pl.ds(ib, WB_ROWS)])

    @pl.when(s == 0)
    def _():
        pltpu.sync_copy(x_hbm.at[pl.ds(rem0, WB_REM)],
                        acc_sh.at[pl.ds(rem0, WB_REM)])

    plsc.subcore_barrier()
    if _SC_TEST == 4:
        plsc.subcore_barrier()

    @pl.when(c == 0)
    def _():
        pltpu.sync_copy(acc_sh.at[pl.ds(ib, WB_ROWS)],
                        out_hbm.at[pl.ds(ib, WB_ROWS)])

        @pl.when(s == 0)
        def _():
            pltpu.sync_copy(acc_sh.at[pl.ds(rem0, WB_REM)],
                            out_hbm.at[pl.ds(rem0, WB_REM)])


def _make_msg(W):
    out_type = (jax.ShapeDtypeStruct((N, W), jnp.float32),
                jax.ShapeDtypeStruct((N, W), jnp.float32))
    scratch = [
        pltpu.VMEM((MSG_CHUNKS_PER_TILE, CH), jnp.int32),
        pltpu.VMEM((CH,), jnp.int32),
        pltpu.VMEM((CH,), jnp.int32),
        pltpu.VMEM((CH, W), jnp.float32),
        pltpu.VMEM((CH, W), jnp.float32),
        pltpu.VMEM_SHARED((ACC_ROWS, W), jnp.float32),
        pltpu.SemaphoreType.DMA,
        pltpu.SemaphoreType.DMA,
        pltpu.SemaphoreType.DMA,
        pltpu.SemaphoreType.DMA,
    ]

    @functools.partial(pl.kernel, out_type=out_type, mesh=_mesh(),
                       scratch_types=scratch)
    def msg(x_lo, x_hi, src_hbm, dst_hbm, z_acc, out_lo, out_hi,
            src_t, dst_c0, dst_c1, rows_v0, rows_v1, acc_sh,
            gsem0, gsem1, dsem0, dsem1):
        c = lax.axis_index("c")
        s = lax.axis_index("s")

        ib = s * WB_ROWS
        rem0 = NTILES * WB_ROWS
        nrem = ACC_ROWS - rem0
        pltpu.sync_copy(z_acc.at[pl.ds(ib, WB_ROWS)],
                        acc_sh.at[pl.ds(ib, WB_ROWS)])

        @pl.when(s == 0)
        def _():
            pltpu.sync_copy(z_acc.at[pl.ds(rem0, nrem)],
                            acc_sh.at[pl.ds(rem0, nrem)])

        base = s * MSG_CHUNKS_PER_TILE
        plsc.subcore_barrier()

        rows = (rows_v0, rows_v1)
        dstc = (dst_c0, dst_c1)
        gsem = (gsem0, gsem1)
        dsem = (dsem0, dsem1)

        def run(x_ref):
            pltpu.sync_copy(src_hbm.at[pl.ds(base, MSG_CHUNKS_PER_TILE)],
                            src_t)
            for k in range(2):
                pltpu.async_copy(x_ref.at[src_t.at[k]], rows[k], gsem[k])
                pltpu.async_copy(dst_hbm.at[base + k], dstc[k], dsem[k])

            def group(g, carry):
                for k in range(2):
                    i = g * 2 + k
                    pltpu.make_async_copy(
                        x_ref.at[src_t.at[0]], rows[k], gsem[k]).wait()
                    pltpu.make_async_copy(
                        dst_hbm.at[0], dstc[k], dsem[k]).wait()
                    pltpu.sync_copy(rows[k], acc_sh.at[dstc[k]], add=True)

                    @pl.when(i + 2 < MSG_CHUNKS_PER_TILE)
                    def _():
                        pltpu.async_copy(x_ref.at[src_t.at[i + 2]],
                                         rows[k], gsem[k])
                        pltpu.async_copy(dst_hbm.at[base + i + 2],
                                         dstc[k], dsem[k])
                return carry
            lax.fori_loop(0, MSG_CHUNKS_PER_TILE // 2, group, 0)

        @pl.when(c == 0)
        def _():
            run(x_lo)

        @pl.when(c == 1)
        def _():
            run(x_hi)

        plsc.subcore_barrier()

        @pl.when(c == 0)
        def _():
            pltpu.sync_copy(acc_sh.at[pl.ds(ib, WB_ROWS)],
                            out_lo.at[pl.ds(ib, WB_ROWS)])

            @pl.when(s == 0)
            def _():
                pltpu.sync_copy(acc_sh.at[pl.ds(rem0, WB_REM)],
                                out_lo.at[pl.ds(rem0, WB_REM)])

        @pl.when(c == 1)
        def _():
            pltpu.sync_copy(acc_sh.at[pl.ds(ib, WB_ROWS)],
                            out_hi.at[pl.ds(ib, WB_ROWS)])

            @pl.when(s == 0)
            def _():
                pltpu.sync_copy(acc_sh.at[pl.ds(rem0, WB_REM)],
                                out_hi.at[pl.ds(rem0, WB_REM)])

    return msg


_msg_half = _make_msg(HH)


@functools.partial(
    pl.kernel,
    out_type=(jax.ShapeDtypeStruct((N, HH), jnp.float32),
              jax.ShapeDtypeStruct((N, HH), jnp.float32)),
    mesh=_mesh(),
    scratch_types=[
        pltpu.VMEM((CH,), jnp.int32),
        pltpu.VMEM((CH,), jnp.int32),
        pltpu.VMEM((CH, HH), jnp.float32),
        pltpu.VMEM_SHARED((ACC_ROWS, HH), jnp.float32),
        pltpu.SemaphoreType.DMA,
        pltpu.SemaphoreType.DMA,
    ])
def _counts(ones_tbl, dub_hbm, dbu_hbm, z_acc, out_cb, out_cu,
            dst_c0, dst_c1, rows_v, acc_sh, dsem0, dsem1):
    c = lax.axis_index("c")
    s = lax.axis_index("s")
    ib = s * WB_ROWS
    rem0 = NTILES * WB_ROWS
    nrem = ACC_ROWS - rem0
    base = s * MSG_CHUNKS_PER_TILE

    pltpu.sync_copy(z_acc.at[pl.ds(ib, WB_ROWS)],
                    acc_sh.at[pl.ds(ib, WB_ROWS)])

    @pl.when(s == 0)
    def _():
        pltpu.sync_copy(z_acc.at[pl.ds(rem0, nrem)],
                        acc_sh.at[pl.ds(rem0, nrem)])
    pltpu.sync_copy(ones_tbl, rows_v)

    plsc.subcore_barrier()

    dstc = (dst_c0, dst_c1)
    dsem = (dsem0, dsem1)

    def run(dst_hbm):
        for k in range(2):
            pltpu.async_copy(dst_hbm.at[base + k], dstc[k], dsem[k])

        def group(g, carry):
            for k in range(2):
                i = g * 2 + k
                pltpu.make_async_copy(
                    dst_hbm.at[0], dstc[k], dsem[k]).wait()
                pltpu.sync_copy(rows_v, acc_sh.at[dstc[k]], add=True)

                @pl.when(i + 2 < MSG_CHUNKS_PER_TILE)
                def _():
                    pltpu.async_copy(dst_hbm.at[base + i + 2],
                                     dstc[k], dsem[k])
            return carry
        lax.fori_loop(0, MSG_CHUNKS_PER_TILE // 2, group, 0)

    @pl.when(c == 0)
    def _():
        run(dub_hbm)

    @pl.when(c == 1)
    def _():
        run(dbu_hbm)

    plsc.subcore_barrier()

    @pl.when(c == 0)
    def _():
        pltpu.sync_copy(acc_sh.at[pl.ds(ib, WB_ROWS)],
                        out_cb.at[pl.ds(ib, WB_ROWS)])

        @pl.when(s == 0)
        def _():
            pltpu.sync_copy(acc_sh.at[pl.ds(rem0, WB_REM)],
                            out_cb.at[pl.ds(rem0, WB_REM)])

    @pl.when(c == 1)
    def _():
        pltpu.sync_copy(acc_sh.at[pl.ds(ib, WB_ROWS)],
                        out_cu.at[pl.ds(ib, WB_ROWS)])

        @pl.when(s == 0)
        def _():
            pltpu.sync_copy(acc_sh.at[pl.ds(rem0, WB_REM)],
                            out_cu.at[pl.ds(rem0, WB_REM)])


@functools.partial(
    pl.kernel,
    out_type=jax.ShapeDtypeStruct((ELP * 16,), jnp.float32),
    mesh=_mesh(),
    scratch_types=[
        pltpu.VMEM((CH,), jnp.int32),
        pltpu.VMEM((CH,), jnp.int32),
        pltpu.VMEM((CH, H), jnp.float32),
        pltpu.VMEM((CH, H), jnp.float32),
        pltpu.VMEM((CH * 16,), jnp.float32),
        pltpu.VMEM((H,), jnp.float32),
        pltpu.SemaphoreType.DMA,
        pltpu.SemaphoreType.DMA,
    ])
def _decoder(pu_hbm, pb_hbm, row_hbm, col_hbm, w2_hbm, out_hbm,
             row_c, col_c, pur_v, pbr_v, part_v, w2_v, sem1, sem2):
    c = lax.axis_index("c")
    s = lax.axis_index("s")
    w = s * 2 + c

    pltpu.sync_copy(w2_hbm, w2_v)
    base = w * DEC_CHUNKS_PER_WORKER

    w2c = [w2_v[pl.ds(16 * j, 16)] for j in range(16)]

    def chunk(i, carry):
        pltpu.sync_copy(row_hbm.at[base + i], row_c)
        pltpu.sync_copy(col_hbm.at[base + i], col_c)
        cp1 = pltpu.async_copy(pu_hbm.at[row_c], pur_v, sem1)
        cp2 = pltpu.async_copy(pb_hbm.at[col_c], pbr_v, sem2)
        cp1.wait()
        cp2.wait()

        def edge(e, carry2):
            acc = jnp.zeros((16,), jnp.float32)
            for j in range(16):
                a = pur_v[e, pl.ds(16 * j, 16)]
                b = pbr_v[e, pl.ds(16 * j, 16)]
                acc = acc + jnp.maximum(a + b, 0.0) * w2c[j]
            part_v[pl.ds(e * 16, 16)] = acc
            return carry2

        lax.fori_loop(0, CH, edge, 0)
        pltpu.sync_copy(part_v, out_hbm.at[pl.ds((base + i) * CH * 16, CH * 16)])
        return carry

    lax.fori_loop(0, DEC_CHUNKS_PER_WORKER, chunk, 0)


_R = 1000


def _dot(a, b):
    return jnp.dot(a, b, preferred_element_type=jnp.float32)


def _proj_body(ux_ref, bx_ref, wu_ref, bu_ref, wb_ref, bb_ref,
               xul, xuh, xbl, xbh):
    xu = _dot(ux_ref[...], wu_ref[...]) + bu_ref[...]
    xb = _dot(bx_ref[...], wb_ref[...]) + bb_ref[...]
    xul[...] = xu[:, :HH]
    xuh[...] = xu[:, HH:]
    xbl[...] = xb[:, :HH]
    xbh[...] = xb[:, HH:]


def _tc_proj(ux, bx, wu, bu, wb, bb):
    return pl.pallas_call(
        _proj_body,
        grid=(N // _R,),
        in_specs=[
            pl.BlockSpec((_R, 8), lambda i: (i, 0)),
            pl.BlockSpec((_R, 384), lambda i: (i, 0)),
            pl.BlockSpec((8, H), lambda i: (0, 0)),
            pl.BlockSpec((1, H), lambda i: (0, 0)),
            pl.BlockSpec((384, H), lambda i: (0, 0)),
            pl.BlockSpec((1, H), lambda i: (0, 0)),
        ],
        out_specs=[pl.BlockSpec((_R, HH), lambda i: (i, 0))] * 4,
        out_shape=[jax.ShapeDtypeStruct((N, HH), jnp.float32)] * 4,
    )(ux, bx, wu, bu, wb, bb)


def _make_combine_body(relu, proj):
    def body(sl, sh, cnt, xl, xh, wll, wlh, wrl, wrh, b, *rest):
        inv = 1.0 / jnp.maximum(cnt[...][:, :1], 1.0)
        h = (_dot(sl[...] * inv, wll[...]) + _dot(sh[...] * inv, wlh[...])
             + _dot(xl[...], wrl[...]) + _dot(xh[...], wrh[...]) + b[...])
        if relu:
            h = jnp.maximum(h, 0.0)
        if proj:
            wp, bp, pout = rest
            pout[...] = _dot(h, wp[...]) + bp[...]
        else:
            ol, oh = rest
            ol[...] = h[:, :HH]
            oh[...] = h[:, HH:]
    return body


_combine1_body = _make_combine_body(relu=True, proj=False)
_combine2_body = _make_combine_body(relu=False, proj=True)

_w_spec = [
    pl.BlockSpec((HH, H), lambda i: (0, 0)),
    pl.BlockSpec((HH, H), lambda i: (0, 0)),
    pl.BlockSpec((HH, H), lambda i: (0, 0)),
    pl.BlockSpec((HH, H), lambda i: (0, 0)),
    pl.BlockSpec((1, H), lambda i: (0, 0)),
]
_node_half_spec = [
    pl.BlockSpec((_R, HH), lambda i: (i, 0)),
    pl.BlockSpec((_R, HH), lambda i: (i, 0)),
    pl.BlockSpec((_R, 16), lambda i: (i, 0)),
    pl.BlockSpec((_R, HH), lambda i: (i, 0)),
    pl.BlockSpec((_R, HH), lambda i: (i, 0)),
]


def _tc_combine1(sl, sh, cnt, xl, xh, wll, wlh, wrl, wrh, b):
    return pl.pallas_call(
        _combine1_body,
        grid=(N // _R,),
        in_specs=_node_half_spec + _w_spec,
        out_specs=[pl.BlockSpec((_R, HH), lambda i: (i, 0))] * 2,
        out_shape=[jax.ShapeDtypeStruct((N, HH), jnp.float32)] * 2,
    )(sl, sh, cnt, xl, xh, wll, wlh, wrl, wrh, b)


def _tc_combine2(sl, sh, cnt, xl, xh, wll, wlh, wrl, wrh, b, wp, bp):
    return pl.pallas_call(
        _combine2_body,
        grid=(N // _R,),
        in_specs=_node_half_spec + _w_spec + [
            pl.BlockSpec((H, H), lambda i: (0, 0)),
            pl.BlockSpec((1, H), lambda i: (0, 0)),
        ],
        out_specs=pl.BlockSpec((_R, H), lambda i: (i, 0)),
        out_shape=jax.ShapeDtypeStruct((N, H), jnp.float32),
    )(sl, sh, cnt, xl, xh, wll, wlh, wrl, wrh, b, wp, bp)


def _tail_body(p_ref, b2_ref, o_ref):
    o_ref[...] = jnp.sum(p_ref[...], axis=1, keepdims=True) + b2_ref[0, 0]


def _tc_tail(parts, b2):
    return pl.pallas_call(
        _tail_body,
        grid=(ELP // 4096,),
        in_specs=[pl.BlockSpec((4096, 16), lambda i: (i, 0)),
                  pl.BlockSpec((1, 1), lambda i: (0, 0),
                               memory_space=pltpu.SMEM)],
        out_specs=pl.BlockSpec((4096, 1), lambda i: (i, 0)),
        out_shape=jax.ShapeDtypeStruct((ELP, 1), jnp.float32),
    )(parts, b2)


def _pad_edges(idx, pad_len, pad_val):
    return jnp.concatenate(
        [idx, jnp.full((pad_len,), pad_val, jnp.int32)]).reshape(-1, CH)


def kernel(user_x, book_x, edge_index_ub, edge_index_bu, edge_label_index,
           user_lin_w, user_lin_b, book_lin_w, book_lin_b,
           c1_ub_wl, c1_ub_wr, c1_ub_b, c1_bu_wl, c1_bu_wr, c1_bu_b,
           c2_ub_wl, c2_ub_wr, c2_ub_b, c2_bu_wl, c2_bu_wr, c2_bu_b,
           dec_w1, dec_b1, dec_w2, dec_b2):
    f32 = jnp.float32
    ux8 = jnp.pad(user_x, ((0, 0), (0, 5)))
    wu8 = jnp.pad(user_lin_w, ((0, 5), (0, 0)))
    src_ub = _pad_edges(edge_index_ub[0], EP - E, 0)
    dst_ub = _pad_edges(edge_index_ub[1], EP - E, N)
    src_bu = _pad_edges(edge_index_bu[0], EP - E, 0)
    dst_bu = _pad_edges(edge_index_bu[1], EP - E, N)
    row_l = _pad_edges(edge_label_index[0], ELP - EL, 0)
    col_l = _pad_edges(edge_label_index[1], ELP - EL, 0)
    z_acc = jnp.zeros((ACC_ROWS, HH), f32)
    ones_tbl = jnp.zeros((CH, HH), f32).at[:, 0].set(1.0)
    w1u = dec_w1[:H]
    w1b = dec_w1[H:]
    b1r = dec_b1.reshape(1, H)
    zb = jnp.zeros((1, H), f32)

    def split(w):
        return w[:HH], w[HH:]

    xu_lo, xu_hi, xb_lo, xb_hi = _tc_proj(
        ux8, book_x, wu8, user_lin_b.reshape(1, H),
        book_lin_w, book_lin_b.reshape(1, H))
    if _SC_TEST == 1:
        xu_lo = _sc_ident(xu_lo)
    elif _SC_TEST in (2, 4):
        xu_lo = _sc_ident2(xu_lo)

    if _DEBUG_XLA_MSG:
        def _xla_msg(xl, xh, src, dst, want_counts):
            s = jax.ops.segment_sum(
                jnp.concatenate([xl, xh], axis=1)[src.reshape(-1)],
                dst.reshape(-1), num_segments=ACC_ROWS)
            outs = [s[:N, :HH], s[:N, HH:]]
            if want_counts:
                cnt = jax.ops.segment_sum(
                    jnp.ones((EP,), f32), dst.reshape(-1),
                    num_segments=ACC_ROWS)
                outs.append(jnp.broadcast_to(cnt[:N, None], (N, 16)))
            return outs
        sb_lo, sb_hi, cb = _xla_msg(xu_lo, xu_hi, src_ub, dst_ub, True)
        su_lo, su_hi, cu = _xla_msg(xb_lo, xb_hi, src_bu, dst_bu, True)
    else:
        sb_lo, sb_hi = _msg_half(xu_lo, xu_hi, src_ub, dst_ub, z_acc)
        su_lo, su_hi = _msg_half(xb_lo, xb_hi, src_bu, dst_bu, z_acc)
        cb128, cu128 = _counts(ones_tbl, dst_ub, dst_bu, z_acc)
        cb, cu = cb128[:, :16], cu128[:, :16]
    wll, wlh = split(c1_ub_wl)
    wrl, wrh = split(c1_ub_wr)
    xb1_lo, xb1_hi = _tc_combine1(sb_lo, sb_hi, cb, xb_lo, xb_hi,
                                  wll, wlh, wrl, wrh, c1_ub_b.reshape(1, H))
    wll, wlh = split(c1_bu_wl)
    wrl, wrh = split(c1_bu_wr)
    xu1_lo, xu1_hi = _tc_combine1(su_lo, su_hi, cu, xu_lo, xu_hi,
                                  wll, wlh, wrl, wrh, c1_bu_b.reshape(1, H))

    if _DEBUG_XLA_MSG:
        sb2_lo, sb2_hi = _xla_msg(xu1_lo, xu1_hi, src_ub, dst_ub, False)
        su2_lo, su2_hi = _xla_msg(xb1_lo, xb1_hi, src_bu, dst_bu, False)
    else:
        sb2_lo, sb2_hi = _msg_half(xu1_lo, xu1_hi, src_ub, dst_ub, z_acc)
        su2_lo, su2_hi = _msg_half(xb1_lo, xb1_hi, src_bu, dst_bu, z_acc)
    wll, wlh = split(c2_ub_wl)
    wrl, wrh = split(c2_ub_wr)
    pb = _tc_combine2(sb2_lo, sb2_hi, cb, xb1_lo, xb1_hi,
                      wll, wlh, wrl, wrh, c2_ub_b.reshape(1, H), w1b, b1r)
    wll, wlh = split(c2_bu_wl)
    wrl, wrh = split(c2_bu_wr)
    pu = _tc_combine2(su2_lo, su2_hi, cu, xu1_lo, xu1_hi,
                      wll, wlh, wrl, wrh, c2_bu_b.reshape(1, H), w1u, zb)

    if _DEBUG_XLA_DECODER:
        z = jax.nn.relu(jnp.take(pu, edge_label_index[0], axis=0)
                        + jnp.take(pb, edge_label_index[1], axis=0))
        return (z @ dec_w2 + dec_b2).reshape(-1)
    parts = _decoder(pu, pb, row_l, col_l, dec_w2.reshape(H))
    out = _tc_tail(parts.reshape(ELP, 16), dec_b2.reshape(1, 1))
    return out.reshape(-1)[:EL]


# speedup vs baseline: 1.4020x; 1.0004x over previous
import functools

import jax
import jax.numpy as jnp
from jax import lax
from jax.experimental import pallas as pl
from jax.experimental.pallas import tpu as pltpu
from jax.experimental.pallas import tpu_sc as plsc

N = 10000
H = 256
HH = 128
E = 160000
EL = 160000
CH = 128
NTILES = 16
NWORK = 32

EP = 163840
MSG_CHUNKS_PER_TILE = 80
ELP = 163840
DEC_CHUNKS_PER_WORKER = 40
ACC_ROWS = N + 8

WB_ROWS = 624
WB_REM = N - NTILES * WB_ROWS


def _mesh():
    return plsc.VectorSubcoreMesh(core_axis_name="c", subcore_axis_name="s")


def _make_msg(W):
    out_type = (jax.ShapeDtypeStruct((N, W), jnp.float32),
                jax.ShapeDtypeStruct((N, W), jnp.float32))
    scratch = [
        pltpu.VMEM((MSG_CHUNKS_PER_TILE, CH), jnp.int32),
        pltpu.VMEM((CH,), jnp.int32),
        pltpu.VMEM((CH,), jnp.int32),
        pltpu.VMEM((CH, W), jnp.float32),
        pltpu.VMEM((CH, W), jnp.float32),
        pltpu.VMEM_SHARED((ACC_ROWS, W), jnp.float32),
        pltpu.SemaphoreType.DMA,
        pltpu.SemaphoreType.DMA,
        pltpu.SemaphoreType.DMA,
        pltpu.SemaphoreType.DMA,
    ]

    @functools.partial(pl.kernel, out_type=out_type, mesh=_mesh(),
                       scratch_types=scratch)
    def msg(x_lo, x_hi, src_hbm, dst_hbm, z_acc, out_lo, out_hi,
            src_t, dst_c0, dst_c1, rows_v0, rows_v1, acc_sh,
            gsem0, gsem1, dsem0, dsem1):
        c = lax.axis_index("c")
        s = lax.axis_index("s")

        ib = s * WB_ROWS
        rem0 = NTILES * WB_ROWS
        nrem = ACC_ROWS - rem0
        pltpu.sync_copy(z_acc.at[pl.ds(ib, WB_ROWS)],
                        acc_sh.at[pl.ds(ib, WB_ROWS)])

        @pl.when(s == 0)
        def _():
            pltpu.sync_copy(z_acc.at[pl.ds(rem0, nrem)],
                            acc_sh.at[pl.ds(rem0, nrem)])

        base = s * MSG_CHUNKS_PER_TILE
        plsc.subcore_barrier()

        rows = (rows_v0, rows_v1)
        dstc = (dst_c0, dst_c1)
        gsem = (gsem0, gsem1)
        dsem = (dsem0, dsem1)

        def run(x_ref):
            pltpu.sync_copy(src_hbm.at[pl.ds(base, MSG_CHUNKS_PER_TILE)],
                            src_t)
            for k in range(2):
                pltpu.async_copy(x_ref.at[src_t.at[k]], rows[k], gsem[k])
                pltpu.async_copy(dst_hbm.at[base + k], dstc[k], dsem[k])

            def group(g, carry):
                for k in range(2):
                    i = g * 2 + k
                    pltpu.make_async_copy(
                        x_ref.at[src_t.at[0]], rows[k], gsem[k]).wait()
                    pltpu.make_async_copy(
                        dst_hbm.at[0], dstc[k], dsem[k]).wait()
                    pltpu.sync_copy(rows[k], acc_sh.at[dstc[k]], add=True)

                    @pl.when(i + 2 < MSG_CHUNKS_PER_TILE)
                    def _():
                        pltpu.async_copy(x_ref.at[src_t.at[i + 2]],
                                         rows[k], gsem[k])
                        pltpu.async_copy(dst_hbm.at[base + i + 2],
                                         dstc[k], dsem[k])
                return carry
            lax.fori_loop(0, MSG_CHUNKS_PER_TILE // 2, group, 0)

        @pl.when(c == 0)
        def _():
            run(x_lo)

        @pl.when(c == 1)
        def _():
            run(x_hi)

        plsc.subcore_barrier()

        @pl.when(c == 0)
        def _():
            pltpu.sync_copy(acc_sh.at[pl.ds(ib, WB_ROWS)],
                            out_lo.at[pl.ds(ib, WB_ROWS)])

            @pl.when(s == 0)
            def _():
                pltpu.sync_copy(acc_sh.at[pl.ds(rem0, WB_REM)],
                                out_lo.at[pl.ds(rem0, WB_REM)])

        @pl.when(c == 1)
        def _():
            pltpu.sync_copy(acc_sh.at[pl.ds(ib, WB_ROWS)],
                            out_hi.at[pl.ds(ib, WB_ROWS)])

            @pl.when(s == 0)
            def _():
                pltpu.sync_copy(acc_sh.at[pl.ds(rem0, WB_REM)],
                                out_hi.at[pl.ds(rem0, WB_REM)])

    return msg


_msg_half = _make_msg(HH)


@functools.partial(
    pl.kernel,
    out_type=(jax.ShapeDtypeStruct((N, HH), jnp.float32),
              jax.ShapeDtypeStruct((N, HH), jnp.float32)),
    mesh=_mesh(),
    scratch_types=[
        pltpu.VMEM((CH,), jnp.int32),
        pltpu.VMEM((CH,), jnp.int32),
        pltpu.VMEM((CH, HH), jnp.float32),
        pltpu.VMEM_SHARED((ACC_ROWS, HH), jnp.float32),
        pltpu.SemaphoreType.DMA,
        pltpu.SemaphoreType.DMA,
    ])
def _counts(ones_tbl, dub_hbm, dbu_hbm, z_acc, out_cb, out_cu,
            dst_c0, dst_c1, rows_v, acc_sh, dsem0, dsem1):
    c = lax.axis_index("c")
    s = lax.axis_index("s")
    ib = s * WB_ROWS
    rem0 = NTILES * WB_ROWS
    nrem = ACC_ROWS - rem0
    base = s * MSG_CHUNKS_PER_TILE

    pltpu.sync_copy(z_acc.at[pl.ds(ib, WB_ROWS)],
                    acc_sh.at[pl.ds(ib, WB_ROWS)])

    @pl.when(s == 0)
    def _():
        pltpu.sync_copy(z_acc.at[pl.ds(rem0, nrem)],
                        acc_sh.at[pl.ds(rem0, nrem)])
    pltpu.sync_copy(ones_tbl, rows_v)

    plsc.subcore_barrier()

    dstc = (dst_c0, dst_c1)
    dsem = (dsem0, dsem1)

    def run(dst_hbm):
        for k in range(2):
            pltpu.async_copy(dst_hbm.at[base + k], dstc[k], dsem[k])

        def group(g, carry):
            for k in range(2):
                i = g * 2 + k
                pltpu.make_async_copy(
                    dst_hbm.at[0], dstc[k], dsem[k]).wait()
                pltpu.sync_copy(rows_v, acc_sh.at[dstc[k]], add=True)

                @pl.when(i + 2 < MSG_CHUNKS_PER_TILE)
                def _():
                    pltpu.async_copy(dst_hbm.at[base + i + 2],
                                     dstc[k], dsem[k])
            return carry
        lax.fori_loop(0, MSG_CHUNKS_PER_TILE // 2, group, 0)

    @pl.when(c == 0)
    def _():
        run(dub_hbm)

    @pl.when(c == 1)
    def _():
        run(dbu_hbm)

    plsc.subcore_barrier()

    @pl.when(c == 0)
    def _():
        pltpu.sync_copy(acc_sh.at[pl.ds(ib, WB_ROWS)],
                        out_cb.at[pl.ds(ib, WB_ROWS)])

        @pl.when(s == 0)
        def _():
            pltpu.sync_copy(acc_sh.at[pl.ds(rem0, WB_REM)],
                            out_cb.at[pl.ds(rem0, WB_REM)])

    @pl.when(c == 1)
    def _():
        pltpu.sync_copy(acc_sh.at[pl.ds(ib, WB_ROWS)],
                        out_cu.at[pl.ds(ib, WB_ROWS)])

        @pl.when(s == 0)
        def _():
            pltpu.sync_copy(acc_sh.at[pl.ds(rem0, WB_REM)],
                            out_cu.at[pl.ds(rem0, WB_REM)])


@functools.partial(
    pl.kernel,
    out_type=jax.ShapeDtypeStruct((ELP * 16,), jnp.float32),
    mesh=_mesh(),
    scratch_types=[
        pltpu.VMEM((CH,), jnp.int32),
        pltpu.VMEM((CH,), jnp.int32),
        pltpu.VMEM((CH, H), jnp.float32),
        pltpu.VMEM((CH, H), jnp.float32),
        pltpu.VMEM((CH * 16,), jnp.float32),
        pltpu.VMEM((H,), jnp.float32),
        pltpu.SemaphoreType.DMA,
        pltpu.SemaphoreType.DMA,
    ])
def _decoder(pu_hbm, pb_hbm, row_hbm, col_hbm, w2_hbm, out_hbm,
             row_c, col_c, pur_v, pbr_v, part_v, w2_v, sem1, sem2):
    c = lax.axis_index("c")
    s = lax.axis_index("s")
    w = s * 2 + c

    pltpu.sync_copy(w2_hbm, w2_v)
    base = w * DEC_CHUNKS_PER_WORKER

    w2c = [w2_v[pl.ds(16 * j, 16)] for j in range(16)]

    def chunk(i, carry):
        pltpu.sync_copy(row_hbm.at[base + i], row_c)
        pltpu.sync_copy(col_hbm.at[base + i], col_c)
        cp1 = pltpu.async_copy(pu_hbm.at[row_c], pur_v, sem1)
        cp2 = pltpu.async_copy(pb_hbm.at[col_c], pbr_v, sem2)
        cp1.wait()
        cp2.wait()

        def edge(e, carry2):
            acc = jnp.zeros((16,), jnp.float32)
            for j in range(16):
                a = pur_v[e, pl.ds(16 * j, 16)]
                b = pbr_v[e, pl.ds(16 * j, 16)]
                acc = acc + jnp.maximum(a + b, 0.0) * w2c[j]
            part_v[pl.ds(e * 16, 16)] = acc
            return carry2

        lax.fori_loop(0, CH, edge, 0)
        pltpu.sync_copy(part_v, out_hbm.at[pl.ds((base + i) * CH * 16, CH * 16)])
        return carry

    lax.fori_loop(0, DEC_CHUNKS_PER_WORKER, chunk, 0)


_R = 1000


def _dot(a, b):
    return jnp.dot(a, b, preferred_element_type=jnp.float32)


def _proj_body(ux_ref, bx_ref, wu_ref, bu_ref, wb_ref, bb_ref,
               xul, xuh, xbl, xbh):
    xu = _dot(ux_ref[...], wu_ref[...]) + bu_ref[...]
    xb = _dot(bx_ref[...], wb_ref[...]) + bb_ref[...]
    xul[...] = xu[:, :HH]
    xuh[...] = xu[:, HH:]
    xbl[...] = xb[:, :HH]
    xbh[...] = xb[:, HH:]


def _tc_proj(ux, bx, wu, bu, wb, bb):
    return pl.pallas_call(
        _proj_body,
        grid=(N // _R,),
        in_specs=[
            pl.BlockSpec((_R, 8), lambda i: (i, 0)),
            pl.BlockSpec((_R, 384), lambda i: (i, 0)),
            pl.BlockSpec((8, H), lambda i: (0, 0)),
            pl.BlockSpec((1, H), lambda i: (0, 0)),
            pl.BlockSpec((384, H), lambda i: (0, 0)),
            pl.BlockSpec((1, H), lambda i: (0, 0)),
        ],
        out_specs=[pl.BlockSpec((_R, HH), lambda i: (i, 0))] * 4,
        out_shape=[jax.ShapeDtypeStruct((N, HH), jnp.float32)] * 4,
    )(ux, bx, wu, bu, wb, bb)


def _make_combine_body(relu, proj):
    def body(sl, sh, cnt, xl, xh, wll, wlh, wrl, wrh, b, *rest):
        inv = 1.0 / jnp.maximum(cnt[...][:, :1], 1.0)
        h = (_dot(sl[...] * inv, wll[...]) + _dot(sh[...] * inv, wlh[...])
             + _dot(xl[...], wrl[...]) + _dot(xh[...], wrh[...]) + b[...])
        if relu:
            h = jnp.maximum(h, 0.0)
        if proj:
            wp, bp, pout = rest
            pout[...] = _dot(h, wp[...]) + bp[...]
        else:
            ol, oh = rest
            ol[...] = h[:, :HH]
            oh[...] = h[:, HH:]
    return body


_combine1_body = _make_combine_body(relu=True, proj=False)
_combine2_body = _make_combine_body(relu=False, proj=True)

_w_spec = [
    pl.BlockSpec((HH, H), lambda i: (0, 0)),
    pl.BlockSpec((HH, H), lambda i: (0, 0)),
    pl.BlockSpec((HH, H), lambda i: (0, 0)),
    pl.BlockSpec((HH, H), lambda i: (0, 0)),
    pl.BlockSpec((1, H), lambda i: (0, 0)),
]
_node_half_spec = [
    pl.BlockSpec((_R, HH), lambda i: (i, 0)),
    pl.BlockSpec((_R, HH), lambda i: (i, 0)),
    pl.BlockSpec((_R, 16), lambda i: (i, 0)),
    pl.BlockSpec((_R, HH), lambda i: (i, 0)),
    pl.BlockSpec((_R, HH), lambda i: (i, 0)),
]


def _tc_combine1(sl, sh, cnt, xl, xh, wll, wlh, wrl, wrh, b):
    return pl.pallas_call(
        _combine1_body,
        grid=(N // _R,),
        in_specs=_node_half_spec + _w_spec,
        out_specs=[pl.BlockSpec((_R, HH), lambda i: (i, 0))] * 2,
        out_shape=[jax.ShapeDtypeStruct((N, HH), jnp.float32)] * 2,
    )(sl, sh, cnt, xl, xh, wll, wlh, wrl, wrh, b)


def _tc_combine2(sl, sh, cnt, xl, xh, wll, wlh, wrl, wrh, b, wp, bp):
    return pl.pallas_call(
        _combine2_body,
        grid=(N // _R,),
        in_specs=_node_half_spec + _w_spec + [
            pl.BlockSpec((H, H), lambda i: (0, 0)),
            pl.BlockSpec((1, H), lambda i: (0, 0)),
        ],
        out_specs=pl.BlockSpec((_R, H), lambda i: (i, 0)),
        out_shape=jax.ShapeDtypeStruct((N, H), jnp.float32),
    )(sl, sh, cnt, xl, xh, wll, wlh, wrl, wrh, b, wp, bp)


def _tail_body(p_ref, b2_ref, o_ref):
    o_ref[...] = jnp.sum(p_ref[...], axis=1, keepdims=True) + b2_ref[0, 0]


def _tc_tail(parts, b2):
    return pl.pallas_call(
        _tail_body,
        grid=(ELP // 4096,),
        in_specs=[pl.BlockSpec((4096, 16), lambda i: (i, 0)),
                  pl.BlockSpec((1, 1), lambda i: (0, 0),
                               memory_space=pltpu.SMEM)],
        out_specs=pl.BlockSpec((4096, 1), lambda i: (i, 0)),
        out_shape=jax.ShapeDtypeStruct((ELP, 1), jnp.float32),
    )(parts, b2)


def _pad_edges(idx, pad_len, pad_val):
    return jnp.concatenate(
        [idx, jnp.full((pad_len,), pad_val, jnp.int32)]).reshape(-1, CH)


def kernel(user_x, book_x, edge_index_ub, edge_index_bu, edge_label_index,
           user_lin_w, user_lin_b, book_lin_w, book_lin_b,
           c1_ub_wl, c1_ub_wr, c1_ub_b, c1_bu_wl, c1_bu_wr, c1_bu_b,
           c2_ub_wl, c2_ub_wr, c2_ub_b, c2_bu_wl, c2_bu_wr, c2_bu_b,
           dec_w1, dec_b1, dec_w2, dec_b2):
    f32 = jnp.float32
    ux8 = jnp.pad(user_x, ((0, 0), (0, 5)))
    wu8 = jnp.pad(user_lin_w, ((0, 5), (0, 0)))
    src_ub = _pad_edges(edge_index_ub[0], EP - E, 0)
    dst_ub = _pad_edges(edge_index_ub[1], EP - E, N)
    src_bu = _pad_edges(edge_index_bu[0], EP - E, 0)
    dst_bu = _pad_edges(edge_index_bu[1], EP - E, N)
    row_l = _pad_edges(edge_label_index[0], ELP - EL, 0)
    col_l = _pad_edges(edge_label_index[1], ELP - EL, 0)
    z_acc = jnp.zeros((ACC_ROWS, HH), f32)
    ones_tbl = jnp.zeros((CH, HH), f32).at[:, 0].set(1.0)
    w1u = dec_w1[:H]
    w1b = dec_w1[H:]
    b1r = dec_b1.reshape(1, H)
    zb = jnp.zeros((1, H), f32)

    def split(w):
        return w[:HH], w[HH:]

    xu_lo, xu_hi, xb_lo, xb_hi = _tc_proj(
        ux8, book_x, wu8, user_lin_b.reshape(1, H),
        book_lin_w, book_lin_b.reshape(1, H))

    sb_lo, sb_hi = _msg_half(xu_lo, xu_hi, src_ub, dst_ub, z_acc)
    su_lo, su_hi = _msg_half(xb_lo, xb_hi, src_bu, dst_bu, z_acc)
    cb128, cu128 = _counts(ones_tbl, dst_ub, dst_bu, z_acc)
    cb, cu = cb128[:, :16], cu128[:, :16]
    wll, wlh = split(c1_ub_wl)
    wrl, wrh = split(c1_ub_wr)
    xb1_lo, xb1_hi = _tc_combine1(sb_lo, sb_hi, cb, xb_lo, xb_hi,
                                  wll, wlh, wrl, wrh, c1_ub_b.reshape(1, H))
    wll, wlh = split(c1_bu_wl)
    wrl, wrh = split(c1_bu_wr)
    xu1_lo, xu1_hi = _tc_combine1(su_lo, su_hi, cu, xu_lo, xu_hi,
                                  wll, wlh, wrl, wrh, c1_bu_b.reshape(1, H))

    sb2_lo, sb2_hi = _msg_half(xu1_lo, xu1_hi, src_ub, dst_ub, z_acc)
    su2_lo, su2_hi = _msg_half(xb1_lo, xb1_hi, src_bu, dst_bu, z_acc)
    wll, wlh = split(c2_ub_wl)
    wrl, wrh = split(c2_ub_wr)
    pb = _tc_combine2(sb2_lo, sb2_hi, cb, xb1_lo, xb1_hi,
                      wll, wlh, wrl, wrh, c2_ub_b.reshape(1, H), w1b, b1r)
    wll, wlh = split(c2_bu_wl)
    wrl, wrh = split(c2_bu_wr)
    pu = _tc_combine2(su2_lo, su2_hi, cu, xu1_lo, xu1_hi,
                      wll, wlh, wrl, wrh, c2_bu_b.reshape(1, H), w1u, zb)

    parts = _decoder(pu, pb, row_l, col_l, dec_w2.reshape(H))
    out = _tc_tail(parts.reshape(ELP, 16), dec_b2.reshape(1, 1))
    return out.reshape(-1)[:EL]
